# baseline (speedup 1.0000x reference)
import functools

import jax
import jax.numpy as jnp
from jax import lax
from jax.experimental import pallas as pl
from jax.experimental.pallas import tpu as pltpu
from jax.experimental.pallas import tpu_sc as plsc

B = 4096
D = 64
DW = D // 2
NNEG = 50
TEMP = 0.1
NC = 2
NS = 16
NW = NC * NS
BPW = B // NW
L = 16
NEG_GROUPS = (NNEG + L - 1) // L
PAIRS = BPW // 2


@functools.partial(
    pl.kernel,
    out_type=(
        jax.ShapeDtypeStruct((NW * BPW // L, L), jnp.float32),
        jax.ShapeDtypeStruct((NW * BPW // L, L), jnp.float32),
    ),
    mesh=plsc.VectorSubcoreMesh(core_axis_name="c", subcore_axis_name="s"),
    compiler_params=pltpu.CompilerParams(use_tc_tiling_on_sc=False,
                                         needs_layout_passes=False),
    scratch_types=[
        pltpu.VMEM((BPW,), jnp.int32),
        pltpu.VMEM((BPW,), jnp.int32),
        pltpu.VMEM((PAIRS, 2 * NNEG), jnp.int32),
        pltpu.VMEM((BPW, D), jnp.bfloat16),
        pltpu.VMEM((BPW, D), jnp.bfloat16),
        pltpu.VMEM((2 * NNEG, D), jnp.bfloat16),
        pltpu.VMEM((2 * NNEG, D), jnp.bfloat16),
        pltpu.VMEM((BPW // L, L), jnp.float32),
        pltpu.VMEM((BPW // L, L), jnp.float32),
        pltpu.SemaphoreType.DMA,
        pltpu.SemaphoreType.DMA,
        pltpu.SemaphoreType.DMA,
    ],
)
def _sc_scores(users_hbm, pos_hbm, neg_hbm, uemb_hbm, iemb_hbm,
               pd_out, ns_out,
               uidx, pidx, nidx, urows, prows, nb0, nb1, pd_v, ns_v,
               sem0, sem1, sem2):
    wid = lax.axis_index("s") * NC + lax.axis_index("c")
    base = wid * BPW
    lane = lax.iota(jnp.int32, L)

    pltpu.sync_copy(users_hbm.at[pl.ds(base, BPW)], uidx)
    pltpu.sync_copy(pos_hbm.at[pl.ds(base, BPW)], pidx)
    pltpu.sync_copy(neg_hbm.at[pl.ds(wid * PAIRS, PAIRS)], nidx)

    cu = pltpu.make_async_copy(uemb_hbm.at[uidx], urows, sem2)
    cu.start()
    cp = pltpu.make_async_copy(iemb_hbm.at[pidx], prows, sem2)
    cp.start()
    pltpu.make_async_copy(iemb_hbm.at[nidx.at[0]], nb0, sem0).start()
    pltpu.make_async_copy(iemb_hbm.at[nidx.at[1]], nb1, sem1).start()
    cu.wait()
    cp.wait()

    for r in range(BPW // L):
        pd_v[r] = jnp.zeros((L,), jnp.float32)
        ns_v[r] = jnp.zeros((L,), jnp.float32)

    perms = {w: lane ^ w for w in (8, 4, 2, 1)}
    masks = {w: (lane & w) == 0 for w in (8, 4, 2, 1)}
    gdn = lax.GatherDimensionNumbers(
        offset_dims=(), collapsed_slice_dims=(0,), start_index_map=(0,))

    def _take(v, w):
        return lax.gather(v, perms[w][:, None], dimension_numbers=gdn,
                          slice_sizes=(1,),
                          mode=lax.GatherScatterMode.PROMISE_IN_BOUNDS)

    def _hsum(v):
        for w in (8, 4, 2, 1):
            v = v + _take(v, w)
        return v

    def _butterfly(vecs):
        for w in (8, 4, 2, 1):
            nxt = []
            for i in range(0, len(vecs), 2):
                a, c = vecs[i], vecs[i + 1]
                nxt.append(jnp.where(masks[w], a + _take(a, w), c + _take(c, w)))
            vecs = nxt
        return vecs[0]

    def _row4(ref, r, off):
        w0 = ref[r, pl.ds(off, 2 * L)]
        w1 = ref[r, pl.ds(off + 2 * L, 2 * L)]
        l0, h0 = plsc.unpack(w0, format=plsc.PackFormat.INTERLEAVED)
        l1, h1 = plsc.unpack(w1, format=plsc.PackFormat.INTERLEAVED)
        return l0, h0, l1, h1

    def _dot_parts(u4, ref, r, off):
        l0, h0, l1, h1 = _row4(ref, r, off)
        return u4[0] * l0 + u4[1] * h0 + u4[2] * l1 + u4[3] * h1

    def do_b(b, nb, nb_half):
        u4 = _row4(urows, b, 0)

        pdv = _hsum(_dot_parts(u4, prows, b, 0))

        pad = jnp.full((L,), -1e29, jnp.float32)
        nacc = jnp.zeros((L,), jnp.float32)
        for g in range(NEG_GROUPS):
            cnt = min(L, NNEG - g * L)
            partials = []
            for k in range(L):
                if k < cnt:
                    n = nb_half * NNEG + g * L + k
                    partials.append(_dot_parts(u4, nb, n, 0))
                else:
                    partials.append(pad)
            sv = _butterfly(partials)
            nacc = nacc + jnp.exp(sv / jnp.float32(TEMP))
        nsv = _hsum(nacc)

        row = b // L
        sel = lane == (b % L)
        plsc.addupdate(pd_v.at[row], jnp.where(sel, pdv, jnp.float32(0.0)))
        plsc.addupdate(ns_v.at[row], jnp.where(sel, nsv, jnp.float32(0.0)))

    def do_pair(p, nb, sem):
        pltpu.make_async_copy(iemb_hbm.at[nidx.at[p]], nb, sem).wait()
        do_b(2 * p, nb, 0)
        do_b(2 * p + 1, nb, 1)

        @pl.when(p + 2 < PAIRS)
        def _():
            pltpu.make_async_copy(iemb_hbm.at[nidx.at[p + 2]], nb, sem).start()

    def body(i, carry):
        do_pair(2 * i, nb0, sem0)
        do_pair(2 * i + 1, nb1, sem1)
        return carry

    lax.fori_loop(0, PAIRS // 2, body, 0)

    pltpu.sync_copy(pd_v, pd_out.at[pl.ds(wid * (BPW // L), BPW // L)])
    pltpu.sync_copy(ns_v, ns_out.at[pl.ds(wid * (BPW // L), BPW // L)])


def _tc_finish_body(pd_ref, ns_ref, o_ref):
    s = pd_ref[...] / jnp.float32(TEMP)
    p = jnp.exp(s)
    loss = jnp.log(p + ns_ref[...]) - s
    o_ref[0, 0] = jnp.sum(loss) * jnp.float32(1.0 / B)


_tc_finish = pl.pallas_call(
    _tc_finish_body,
    out_shape=jax.ShapeDtypeStruct((1, 1), jnp.float32),
    out_specs=pl.BlockSpec(memory_space=pltpu.SMEM),
)


def kernel(users, positives, negatives, epoch, user_emb, item_emb):
    del epoch
    users = users.astype(jnp.int32)
    pos_flat = positives.reshape(B).astype(jnp.int32)
    neg_pairs = negatives.astype(jnp.int32).reshape(B // 2, 2 * NNEG)
    pd, ns = _sc_scores(users, pos_flat, neg_pairs,
                        user_emb.astype(jnp.bfloat16),
                        item_emb.astype(jnp.bfloat16))
    out = _tc_finish(pd.reshape(32, 128), ns.reshape(32, 128))
    return out[0, 0]


# speedup vs baseline: 1.3854x; 1.3854x over previous
import functools

import jax
import jax.numpy as jnp
from jax import lax
from jax.experimental import pallas as pl
from jax.experimental.pallas import tpu as pltpu
from jax.experimental.pallas import tpu_sc as plsc

B = 4096
D = 64
DP = 128
NNEG = 50
TEMP = 0.1
NC = 2
NS = 16
NW = NC * NS
BPW = B // NW
L = 16
NEG_GROUPS = (NNEG + L - 1) // L
PAIRS = BPW // 2


@functools.partial(
    pl.kernel,
    out_type=(
        jax.ShapeDtypeStruct((NW * BPW // L, L), jnp.float32),
        jax.ShapeDtypeStruct((NW * BPW // L, L), jnp.float32),
    ),
    mesh=plsc.VectorSubcoreMesh(core_axis_name="c", subcore_axis_name="s"),
    compiler_params=pltpu.CompilerParams(use_tc_tiling_on_sc=False,
                                         needs_layout_passes=False),
    scratch_types=[
        pltpu.VMEM((BPW,), jnp.int32),
        pltpu.VMEM((BPW,), jnp.int32),
        pltpu.VMEM((PAIRS, 2 * NNEG), jnp.int32),
        pltpu.VMEM((BPW, DP), jnp.float32),
        pltpu.VMEM((BPW, DP), jnp.float32),
        pltpu.VMEM((2 * NNEG, DP), jnp.float32),
        pltpu.VMEM((2 * NNEG, DP), jnp.float32),
        pltpu.VMEM((BPW // L, L), jnp.float32),
        pltpu.VMEM((BPW // L, L), jnp.float32),
        pltpu.SemaphoreType.DMA,
        pltpu.SemaphoreType.DMA,
        pltpu.SemaphoreType.DMA,
    ],
)
def _sc_scores(users_hbm, pos_hbm, neg_hbm, uemb_hbm, iemb_hbm,
               pd_out, ns_out,
               uidx, pidx, nidx, urows, prows, nb0, nb1, pd_v, ns_v,
               sem0, sem1, sem2):
    wid = lax.axis_index("s") * NC + lax.axis_index("c")
    base = wid * BPW
    lane = lax.iota(jnp.int32, L)

    pltpu.sync_copy(users_hbm.at[pl.ds(base, BPW)], uidx)
    pltpu.sync_copy(pos_hbm.at[pl.ds(base, BPW)], pidx)
    pltpu.sync_copy(neg_hbm.at[pl.ds(wid * PAIRS, PAIRS)], nidx)

    cu = pltpu.make_async_copy(uemb_hbm.at[uidx], urows, sem2)
    cu.start()
    cp = pltpu.make_async_copy(iemb_hbm.at[pidx], prows, sem2)
    cp.start()
    pltpu.make_async_copy(iemb_hbm.at[nidx.at[0]], nb0, sem0).start()
    pltpu.make_async_copy(iemb_hbm.at[nidx.at[1]], nb1, sem1).start()
    cu.wait()
    cp.wait()

    for r in range(BPW // L):
        pd_v[r] = jnp.zeros((L,), jnp.float32)
        ns_v[r] = jnp.zeros((L,), jnp.float32)

    perms = {w: lane ^ w for w in (8, 4, 2, 1)}
    masks = {w: (lane & w) == 0 for w in (8, 4, 2, 1)}
    gdn = lax.GatherDimensionNumbers(
        offset_dims=(), collapsed_slice_dims=(0,), start_index_map=(0,))

    def _take(v, w):
        return lax.gather(v, perms[w][:, None], dimension_numbers=gdn,
                          slice_sizes=(1,),
                          mode=lax.GatherScatterMode.PROMISE_IN_BOUNDS)

    def _hsum(v):
        for w in (8, 4, 2, 1):
            v = v + _take(v, w)
        return v

    def _butterfly(vecs):
        for w in (8, 4, 2, 1):
            nxt = []
            for i in range(0, len(vecs), 2):
                a, c = vecs[i], vecs[i + 1]
                nxt.append(jnp.where(masks[w], a + _take(a, w), c + _take(c, w)))
            vecs = nxt
        return vecs[0]

    def _dot_parts(u, ref, r):
        dv = u[0] * ref[r, pl.ds(0, L)]
        for j in range(1, D // L):
            dv = dv + u[j] * ref[r, pl.ds(j * L, L)]
        return dv

    def do_b(b, nb, nb_half):
        u = [urows[b, pl.ds(j * L, L)] for j in range(D // L)]

        pdv = _hsum(_dot_parts(u, prows, b))

        pad = jnp.full((L,), -1e29, jnp.float32)
        nacc = jnp.zeros((L,), jnp.float32)
        for g in range(NEG_GROUPS):
            cnt = min(L, NNEG - g * L)
            partials = []
            for k in range(L):
                if k < cnt:
                    partials.append(
                        _dot_parts(u, nb, nb_half * NNEG + g * L + k))
                else:
                    partials.append(pad)
            sv = _butterfly(partials)
            nacc = nacc + jnp.exp(sv / jnp.float32(TEMP))
        nsv = _hsum(nacc)

        row = b // L
        sel = lane == (b % L)
        plsc.addupdate(pd_v.at[row], jnp.where(sel, pdv, jnp.float32(0.0)))
        plsc.addupdate(ns_v.at[row], jnp.where(sel, nsv, jnp.float32(0.0)))

    def do_pair(p, nb, sem):
        pltpu.make_async_copy(iemb_hbm.at[nidx.at[p]], nb, sem).wait()
        do_b(2 * p, nb, 0)
        do_b(2 * p + 1, nb, 1)

        @pl.when(p + 2 < PAIRS)
        def _():
            pltpu.make_async_copy(iemb_hbm.at[nidx.at[p + 2]], nb, sem).start()

    def body(i, carry):
        do_pair(2 * i, nb0, sem0)
        do_pair(2 * i + 1, nb1, sem1)
        return carry

    lax.fori_loop(0, PAIRS // 2, body, 0)

    pltpu.sync_copy(pd_v, pd_out.at[pl.ds(wid * (BPW // L), BPW // L)])
    pltpu.sync_copy(ns_v, ns_out.at[pl.ds(wid * (BPW // L), BPW // L)])


def _tc_finish_body(pd_ref, ns_ref, o_ref):
    s = pd_ref[...] / jnp.float32(TEMP)
    p = jnp.exp(s)
    loss = jnp.log(p + ns_ref[...]) - s
    o_ref[0, 0] = jnp.sum(loss) * jnp.float32(1.0 / B)


_tc_finish = pl.pallas_call(
    _tc_finish_body,
    out_shape=jax.ShapeDtypeStruct((1, 1), jnp.float32),
    out_specs=pl.BlockSpec(memory_space=pltpu.SMEM),
)


def _pad_rows(table):
    return jnp.pad(table, ((0, 0), (0, DP - D)))


def kernel(users, positives, negatives, epoch, user_emb, item_emb):
    del epoch
    users = users.astype(jnp.int32)
    pos_flat = positives.reshape(B).astype(jnp.int32)
    neg_pairs = negatives.astype(jnp.int32).reshape(B // 2, 2 * NNEG)
    pd, ns = _sc_scores(users, pos_flat, neg_pairs,
                        _pad_rows(user_emb), _pad_rows(item_emb))
    out = _tc_finish(pd.reshape(32, 128), ns.reshape(32, 128))
    return out[0, 0]


# speedup vs baseline: 1.7657x; 1.2745x over previous
import functools

import jax
import jax.numpy as jnp
from jax import lax
from jax.experimental import pallas as pl
from jax.experimental.pallas import tpu as pltpu
from jax.experimental.pallas import tpu_sc as plsc

B = 4096
D = 64
DP = 128
NNEG = 50
TEMP = 0.1
NC = 2
NS = 16
NW = NC * NS
BPW = B // NW
L = 16
NEG_GROUPS = (NNEG + L - 1) // L
PAIRS = BPW // 2


@functools.partial(
    pl.kernel,
    out_type=(
        jax.ShapeDtypeStruct((NW * BPW // L, L), jnp.float32),
        jax.ShapeDtypeStruct((NW * BPW // L, L), jnp.float32),
    ),
    mesh=plsc.VectorSubcoreMesh(core_axis_name="c", subcore_axis_name="s"),
    compiler_params=pltpu.CompilerParams(use_tc_tiling_on_sc=False,
                                         needs_layout_passes=False),
    scratch_types=[
        pltpu.VMEM((BPW,), jnp.int32),
        pltpu.VMEM((BPW,), jnp.int32),
        pltpu.VMEM((PAIRS, 2 * NNEG), jnp.int32),
        pltpu.VMEM((BPW, DP), jnp.float32),
        pltpu.VMEM((BPW, DP), jnp.float32),
        pltpu.VMEM((2 * NNEG, DP), jnp.float32),
        pltpu.VMEM((2 * NNEG, DP), jnp.float32),
        pltpu.VMEM((BPW // L, L), jnp.float32),
        pltpu.VMEM((BPW // L, L), jnp.float32),
        pltpu.SemaphoreType.DMA,
        pltpu.SemaphoreType.DMA,
        pltpu.SemaphoreType.DMA,
    ],
)
def _sc_scores(users_hbm, pos_hbm, neg_hbm, uemb_hbm, iemb_hbm,
               pd_out, ns_out,
               uidx, pidx, nidx, urows, prows, nb0, nb1, pd_v, ns_v,
               sem0, sem1, sem2):
    wid = lax.axis_index("s") * NC + lax.axis_index("c")
    base = wid * BPW
    lane = lax.iota(jnp.int32, L)

    pltpu.sync_copy(users_hbm.at[pl.ds(base, BPW)], uidx)
    pltpu.sync_copy(pos_hbm.at[pl.ds(base, BPW)], pidx)
    pltpu.sync_copy(neg_hbm.at[pl.ds(wid * PAIRS, PAIRS)], nidx)

    cu = pltpu.make_async_copy(uemb_hbm.at[uidx], urows, sem2)
    cu.start()
    cp = pltpu.make_async_copy(iemb_hbm.at[pidx], prows, sem2)
    cp.start()
    pltpu.make_async_copy(iemb_hbm.at[nidx.at[0]], nb0, sem0).start()
    pltpu.make_async_copy(iemb_hbm.at[nidx.at[1]], nb1, sem1).start()
    cu.wait()
    cp.wait()

    for r in range(BPW // L):
        pd_v[r] = jnp.zeros((L,), jnp.float32)
        ns_v[r] = jnp.zeros((L,), jnp.float32)

    perms = {w: lane ^ w for w in (8, 4, 2, 1)}
    masks = {w: (lane & w) == 0 for w in (8, 4, 2, 1)}
    gdn = lax.GatherDimensionNumbers(
        offset_dims=(), collapsed_slice_dims=(0,), start_index_map=(0,))

    def _take(v, w):
        return lax.gather(v, perms[w][:, None], dimension_numbers=gdn,
                          slice_sizes=(1,),
                          mode=lax.GatherScatterMode.PROMISE_IN_BOUNDS)

    def _hsum(v):
        for w in (8, 4, 2, 1):
            v = v + _take(v, w)
        return v

    def _butterfly(vecs):
        for w in (8, 4, 2, 1):
            nxt = []
            for i in range(0, len(vecs), 2):
                a, c = vecs[i], vecs[i + 1]
                nxt.append(jnp.where(masks[w], a + _take(a, w), c + _take(c, w)))
            vecs = nxt
        return vecs[0]

    def _dot_parts(u, ref, r):
        dv = u[0] * ref[r, pl.ds(0, L)]
        for j in range(1, D // L):
            dv = dv + u[j] * ref[r, pl.ds(j * L, L)]
        return dv

    def do_b(b, nb, nb_half):
        u = [urows[b, pl.ds(j * L, L)] for j in range(D // L)]

        pdv = _hsum(_dot_parts(u, prows, b))

        pad = jnp.full((L,), -1e29, jnp.float32)
        nacc = jnp.zeros((L,), jnp.float32)
        for g in range(NEG_GROUPS):
            cnt = min(L, NNEG - g * L)
            partials = []
            for k in range(L):
                if k < cnt:
                    partials.append(
                        _dot_parts(u, nb, nb_half * NNEG + g * L + k))
                else:
                    partials.append(pad)
            sv = _butterfly(partials)
            nacc = nacc + jnp.exp(sv / jnp.float32(TEMP))
        nsv = _hsum(nacc)

        row = b // L
        sel = lane == (b % L)
        plsc.addupdate(pd_v.at[row], jnp.where(sel, pdv, jnp.float32(0.0)))
        plsc.addupdate(ns_v.at[row], jnp.where(sel, nsv, jnp.float32(0.0)))

    def do_pair(p, nb, sem):
        pltpu.make_async_copy(iemb_hbm.at[nidx.at[p]], nb, sem).wait()
        do_b(2 * p, nb, 0)
        do_b(2 * p + 1, nb, 1)

        @pl.when(p + 2 < PAIRS)
        def _():
            pltpu.make_async_copy(iemb_hbm.at[nidx.at[p + 2]], nb, sem).start()

    def body(i, carry):
        do_pair(2 * i, nb0, sem0)
        do_pair(2 * i + 1, nb1, sem1)
        return carry

    lax.fori_loop(0, PAIRS // 2, body, 0)

    pltpu.sync_copy(pd_v, pd_out.at[pl.ds(wid * (BPW // L), BPW // L)])
    pltpu.sync_copy(ns_v, ns_out.at[pl.ds(wid * (BPW // L), BPW // L)])


def _tc_finish_body(pd_ref, ns_ref, o_ref):
    s = pd_ref[...] / jnp.float32(TEMP)
    p = jnp.exp(s)
    loss = jnp.log(p + ns_ref[...]) - s
    o_ref[0, 0] = jnp.sum(loss) * jnp.float32(1.0 / B)


_tc_finish = pl.pallas_call(
    _tc_finish_body,
    out_shape=jax.ShapeDtypeStruct((1, 1), jnp.float32),
    out_specs=pl.BlockSpec(memory_space=pltpu.SMEM),
)


TBLK = 4096


def _tp_body(x_ref, o_ref):
    t = jnp.swapaxes(x_ref[...], 0, 1)
    o_ref[...] = jnp.concatenate([t, t], axis=1)


def _relayout_rows(table):
    n = table.shape[0]
    return pl.pallas_call(
        _tp_body,
        grid=((n + TBLK - 1) // TBLK,),
        in_specs=[pl.BlockSpec((D, TBLK), lambda j: (0, j))],
        out_specs=pl.BlockSpec((TBLK, DP), lambda j: (j, 0)),
        out_shape=jax.ShapeDtypeStruct((n, DP), jnp.float32),
    )(table.T)


def kernel(users, positives, negatives, epoch, user_emb, item_emb):
    del epoch
    users = users.astype(jnp.int32)
    pos_flat = positives.reshape(B).astype(jnp.int32)
    neg_pairs = negatives.astype(jnp.int32).reshape(B // 2, 2 * NNEG)
    pd, ns = _sc_scores(users, pos_flat, neg_pairs,
                        _relayout_rows(user_emb), _relayout_rows(item_emb))
    out = _tc_finish(pd.reshape(32, 128), ns.reshape(32, 128))
    return out[0, 0]


# speedup vs baseline: 2.1402x; 1.2121x over previous
import functools

import jax
import jax.numpy as jnp
from jax import lax
from jax.experimental import pallas as pl
from jax.experimental.pallas import tpu as pltpu
from jax.experimental.pallas import tpu_sc as plsc

B = 4096
D = 64
NNEG = 50
TEMP = 0.1
NC = 2
NS = 16
NW = NC * NS
BPW = B // NW
L = 16
NEG_GROUPS = (NNEG + L - 1) // L
PAIRS = BPW // 2


@functools.partial(
    pl.kernel,
    out_type=(
        jax.ShapeDtypeStruct((NW * BPW // L, L), jnp.float32),
        jax.ShapeDtypeStruct((NW * BPW // L, L), jnp.float32),
    ),
    mesh=plsc.VectorSubcoreMesh(core_axis_name="c", subcore_axis_name="s"),
    compiler_params=pltpu.CompilerParams(use_tc_tiling_on_sc=False,
                                         needs_layout_passes=False),
    scratch_types=[
        pltpu.VMEM((BPW,), jnp.int32),
        pltpu.VMEM((BPW,), jnp.int32),
        pltpu.VMEM((PAIRS, 2 * NNEG), jnp.int32),
        pltpu.VMEM((BPW, D), jnp.float32),
        pltpu.VMEM((BPW, D), jnp.float32),
        pltpu.VMEM((2 * NNEG, D), jnp.float32),
        pltpu.VMEM((2 * NNEG, D), jnp.float32),
        pltpu.VMEM((BPW // L, L), jnp.float32),
        pltpu.VMEM((BPW // L, L), jnp.float32),
        pltpu.SemaphoreType.DMA,
        pltpu.SemaphoreType.DMA,
        pltpu.SemaphoreType.DMA,
    ],
)
def _sc_scores(users_hbm, pos_hbm, neg_hbm, uemb_hbm, iemb_hbm,
               pd_out, ns_out,
               uidx, pidx, nidx, urows, prows, nb0, nb1, pd_v, ns_v,
               sem0, sem1, sem2):
    wid = lax.axis_index("s") * NC + lax.axis_index("c")
    base = wid * BPW
    lane = lax.iota(jnp.int32, L)

    pltpu.sync_copy(users_hbm.at[pl.ds(base, BPW)], uidx)
    pltpu.sync_copy(pos_hbm.at[pl.ds(base, BPW)], pidx)
    pltpu.sync_copy(neg_hbm.at[pl.ds(wid * PAIRS, PAIRS)], nidx)

    cu = pltpu.make_async_copy(uemb_hbm.at[uidx], urows, sem2)
    cu.start()
    cp = pltpu.make_async_copy(iemb_hbm.at[pidx], prows, sem2)
    cp.start()
    pltpu.make_async_copy(iemb_hbm.at[nidx.at[0]], nb0, sem0).start()
    pltpu.make_async_copy(iemb_hbm.at[nidx.at[1]], nb1, sem1).start()
    cu.wait()
    cp.wait()

    for r in range(BPW // L):
        pd_v[r] = jnp.zeros((L,), jnp.float32)
        ns_v[r] = jnp.zeros((L,), jnp.float32)

    perms = {w: lane ^ w for w in (8, 4, 2, 1)}
    masks = {w: (lane & w) == 0 for w in (8, 4, 2, 1)}
    gdn = lax.GatherDimensionNumbers(
        offset_dims=(), collapsed_slice_dims=(0,), start_index_map=(0,))

    def _take(v, w):
        return lax.gather(v, perms[w][:, None], dimension_numbers=gdn,
                          slice_sizes=(1,),
                          mode=lax.GatherScatterMode.PROMISE_IN_BOUNDS)

    def _hsum(v):
        for w in (8, 4, 2, 1):
            v = v + _take(v, w)
        return v

    def _butterfly(vecs):
        for w in (8, 4, 2, 1):
            nxt = []
            for i in range(0, len(vecs), 2):
                a, c = vecs[i], vecs[i + 1]
                nxt.append(jnp.where(masks[w], a + _take(a, w), c + _take(c, w)))
            vecs = nxt
        return vecs[0]

    def _dot_parts(u, ref, r):
        dv = u[0] * ref[r, pl.ds(0, L)]
        for j in range(1, D // L):
            dv = dv + u[j] * ref[r, pl.ds(j * L, L)]
        return dv

    def do_b(b, nb, nb_half):
        u = [urows[b, pl.ds(j * L, L)] for j in range(D // L)]

        pdv = _hsum(_dot_parts(u, prows, b))

        pad = jnp.full((L,), -1e29, jnp.float32)
        nacc = jnp.zeros((L,), jnp.float32)
        for g in range(NEG_GROUPS):
            cnt = min(L, NNEG - g * L)
            partials = []
            for k in range(L):
                if k < cnt:
                    partials.append(
                        _dot_parts(u, nb, nb_half * NNEG + g * L + k))
                else:
                    partials.append(pad)
            sv = _butterfly(partials)
            nacc = nacc + jnp.exp(sv / jnp.float32(TEMP))
        nsv = _hsum(nacc)

        row = b // L
        sel = lane == (b % L)
        plsc.addupdate(pd_v.at[row], jnp.where(sel, pdv, jnp.float32(0.0)))
        plsc.addupdate(ns_v.at[row], jnp.where(sel, nsv, jnp.float32(0.0)))

    def do_pair(p, nb, sem):
        pltpu.make_async_copy(iemb_hbm.at[nidx.at[p]], nb, sem).wait()
        do_b(2 * p, nb, 0)
        do_b(2 * p + 1, nb, 1)

        @pl.when(p + 2 < PAIRS)
        def _():
            pltpu.make_async_copy(iemb_hbm.at[nidx.at[p + 2]], nb, sem).start()

    def body(i, carry):
        do_pair(2 * i, nb0, sem0)
        do_pair(2 * i + 1, nb1, sem1)
        return carry

    lax.fori_loop(0, PAIRS // 2, body, 0)

    pltpu.sync_copy(pd_v, pd_out.at[pl.ds(wid * (BPW // L), BPW // L)])
    pltpu.sync_copy(ns_v, ns_out.at[pl.ds(wid * (BPW // L), BPW // L)])


def _tc_finish_body(pd_ref, ns_ref, o_ref):
    s = pd_ref[...] / jnp.float32(TEMP)
    p = jnp.exp(s)
    loss = jnp.log(p + ns_ref[...]) - s
    o_ref[0, 0] = jnp.sum(loss) * jnp.float32(1.0 / B)


_tc_finish = pl.pallas_call(
    _tc_finish_body,
    out_shape=jax.ShapeDtypeStruct((1, 1), jnp.float32),
    out_specs=pl.BlockSpec(memory_space=pltpu.SMEM),
)


TBLK = 4096


def _tp_body(x_ref, o_ref, t_ref):
    t_ref[...] = jnp.swapaxes(x_ref[...], 0, 1)
    e = t_ref[pl.Slice(0, TBLK // 2, 2), :]
    o = t_ref[pl.Slice(1, TBLK // 2, 2), :]
    o_ref[...] = jnp.concatenate([e, o], axis=1)


def _relayout_rows(table):
    n = table.shape[0]
    packed = pl.pallas_call(
        _tp_body,
        grid=((n + TBLK - 1) // TBLK,),
        in_specs=[pl.BlockSpec((D, TBLK), lambda j: (0, j))],
        out_specs=pl.BlockSpec((TBLK // 2, 2 * D), lambda j: (j, 0)),
        out_shape=jax.ShapeDtypeStruct((n // 2, 2 * D), jnp.float32),
        scratch_shapes=[pltpu.VMEM((TBLK, D), jnp.float32)],
    )(table.T)
    return packed.reshape(n, D)


def kernel(users, positives, negatives, epoch, user_emb, item_emb):
    del epoch
    users = users.astype(jnp.int32)
    pos_flat = positives.reshape(B).astype(jnp.int32)
    neg_pairs = negatives.astype(jnp.int32).reshape(B // 2, 2 * NNEG)
    pd, ns = _sc_scores(users, pos_flat, neg_pairs,
                        _relayout_rows(user_emb), _relayout_rows(item_emb))
    out = _tc_finish(pd.reshape(32, 128), ns.reshape(32, 128))
    return out[0, 0]


# speedup vs baseline: 2.5794x; 1.2052x over previous
import functools

import jax
import jax.numpy as jnp
from jax import lax
from jax.experimental import pallas as pl
from jax.experimental.pallas import tpu as pltpu
from jax.experimental.pallas import tpu_sc as plsc

B = 4096
D = 64
NNEG = 50
TEMP = 0.1
NC = 2
NS = 16
NW = NC * NS
BPW = B // NW
L = 16
NEG_GROUPS = (NNEG + L - 1) // L
PAIRS = BPW // 2


@functools.partial(
    pl.kernel,
    out_type=(
        jax.ShapeDtypeStruct((NW * BPW // L, L), jnp.float32),
        jax.ShapeDtypeStruct((NW * BPW // L, L), jnp.float32),
    ),
    mesh=plsc.VectorSubcoreMesh(core_axis_name="c", subcore_axis_name="s"),
    compiler_params=pltpu.CompilerParams(use_tc_tiling_on_sc=False,
                                         needs_layout_passes=False),
    scratch_types=[
        pltpu.VMEM((BPW,), jnp.int32),
        pltpu.VMEM((BPW,), jnp.int32),
        pltpu.VMEM((PAIRS, 2 * NNEG), jnp.int32),
        pltpu.VMEM((BPW, D), jnp.float32),
        pltpu.VMEM((BPW, D), jnp.float32),
        pltpu.VMEM((2 * NNEG, D), jnp.float32),
        pltpu.VMEM((2 * NNEG, D), jnp.float32),
        pltpu.VMEM((BPW // L, L), jnp.float32),
        pltpu.VMEM((BPW // L, L), jnp.float32),
        pltpu.SemaphoreType.DMA,
        pltpu.SemaphoreType.DMA,
        pltpu.SemaphoreType.DMA,
    ],
)
def _sc_scores(users_hbm, pos_hbm, neg_hbm, uemb_hbm, iemb_hbm,
               pd_out, ns_out,
               uidx, pidx, nidx, urows, prows, nb0, nb1, pd_v, ns_v,
               sem0, sem1, sem2):
    wid = lax.axis_index("s") * NC + lax.axis_index("c")
    base = wid * BPW
    lane = lax.iota(jnp.int32, L)

    pltpu.sync_copy(users_hbm.at[pl.ds(base, BPW)], uidx)
    pltpu.sync_copy(pos_hbm.at[pl.ds(base, BPW)], pidx)
    pltpu.sync_copy(neg_hbm.at[pl.ds(wid * PAIRS, PAIRS)], nidx)

    cu = pltpu.make_async_copy(uemb_hbm.at[uidx], urows, sem2)
    cu.start()
    cp = pltpu.make_async_copy(iemb_hbm.at[pidx], prows, sem2)
    cp.start()
    pltpu.make_async_copy(iemb_hbm.at[nidx.at[0]], nb0, sem0).start()
    pltpu.make_async_copy(iemb_hbm.at[nidx.at[1]], nb1, sem1).start()
    cu.wait()
    cp.wait()

    for r in range(BPW // L):
        pd_v[r] = jnp.zeros((L,), jnp.float32)
        ns_v[r] = jnp.zeros((L,), jnp.float32)

    perms = {w: lane ^ w for w in (8, 4, 2, 1)}
    masks = {w: (lane & w) == 0 for w in (8, 4, 2, 1)}
    gdn = lax.GatherDimensionNumbers(
        offset_dims=(), collapsed_slice_dims=(0,), start_index_map=(0,))

    def _take(v, w):
        return lax.gather(v, perms[w][:, None], dimension_numbers=gdn,
                          slice_sizes=(1,),
                          mode=lax.GatherScatterMode.PROMISE_IN_BOUNDS)

    def _hsum(v):
        for w in (8, 4, 2, 1):
            v = v + _take(v, w)
        return v

    def _butterfly(vecs):
        for w in (8, 4, 2, 1):
            nxt = []
            for i in range(0, len(vecs), 2):
                a, c = vecs[i], vecs[i + 1]
                nxt.append(jnp.where(masks[w], a + _take(a, w), c + _take(c, w)))
            vecs = nxt
        return vecs[0]

    def _dot_parts(u, ref, r):
        dv = u[0] * ref[r, pl.ds(0, L)]
        for j in range(1, D // L):
            dv = dv + u[j] * ref[r, pl.ds(j * L, L)]
        return dv

    def do_b(b, nb, nb_half):
        u = [urows[b, pl.ds(j * L, L)] for j in range(D // L)]

        pdv = _hsum(_dot_parts(u, prows, b))

        pad = jnp.full((L,), -1e29, jnp.float32)
        nacc = jnp.zeros((L,), jnp.float32)
        for g in range(NEG_GROUPS):
            cnt = min(L, NNEG - g * L)
            partials = []
            for k in range(L):
                if k < cnt:
                    partials.append(
                        _dot_parts(u, nb, nb_half * NNEG + g * L + k))
                else:
                    partials.append(pad)
            sv = _butterfly(partials)
            nacc = nacc + jnp.exp(sv / jnp.float32(TEMP))
        nsv = _hsum(nacc)

        row = b // L
        sel = lane == (b % L)
        plsc.addupdate(pd_v.at[row], jnp.where(sel, pdv, jnp.float32(0.0)))
        plsc.addupdate(ns_v.at[row], jnp.where(sel, nsv, jnp.float32(0.0)))

    def do_pair(p, nb, sem):
        pltpu.make_async_copy(iemb_hbm.at[nidx.at[p]], nb, sem).wait()
        do_b(2 * p, nb, 0)
        do_b(2 * p + 1, nb, 1)

        @pl.when(p + 2 < PAIRS)
        def _():
            pltpu.make_async_copy(iemb_hbm.at[nidx.at[p + 2]], nb, sem).start()

    def body(i, carry):
        do_pair(2 * i, nb0, sem0)
        do_pair(2 * i + 1, nb1, sem1)
        return carry

    lax.fori_loop(0, PAIRS // 2, body, 0)

    pltpu.sync_copy(pd_v, pd_out.at[pl.ds(wid * (BPW // L), BPW // L)])
    pltpu.sync_copy(ns_v, ns_out.at[pl.ds(wid * (BPW // L), BPW // L)])


def _tc_finish_body(pd_ref, ns_ref, o_ref):
    s = pd_ref[...] / jnp.float32(TEMP)
    p = jnp.exp(s)
    loss = jnp.log(p + ns_ref[...]) - s
    o_ref[0, 0] = jnp.sum(loss) * jnp.float32(1.0 / B)


_tc_finish = pl.pallas_call(
    _tc_finish_body,
    out_shape=jax.ShapeDtypeStruct((1, 1), jnp.float32),
    out_specs=pl.BlockSpec(memory_space=pltpu.SMEM),
)


TBLK = 8192


def _tp_body(x_ref, o_ref, t_ref):
    t_ref[...] = jnp.swapaxes(x_ref[...], 0, 1)
    e = t_ref[pl.Slice(0, TBLK // 2, 2), :]
    o = t_ref[pl.Slice(1, TBLK // 2, 2), :]
    o_ref[...] = jnp.concatenate([e, o], axis=1)


def _relayout_rows(table):
    n = table.shape[0]
    packed = pl.pallas_call(
        _tp_body,
        grid=((n + TBLK - 1) // TBLK,),
        in_specs=[pl.BlockSpec((D, TBLK), lambda j: (0, j))],
        out_specs=pl.BlockSpec((TBLK // 2, 2 * D), lambda j: (j, 0)),
        out_shape=jax.ShapeDtypeStruct((n // 2, 2 * D), jnp.float32),
        scratch_shapes=[pltpu.VMEM((TBLK, D), jnp.float32)],
    )(table.T)
    return packed.reshape(n, D)


def kernel(users, positives, negatives, epoch, user_emb, item_emb):
    del epoch
    users = users.astype(jnp.int32)
    pos_flat = positives.reshape(B).astype(jnp.int32)
    neg_pairs = negatives.astype(jnp.int32).reshape(B // 2, 2 * NNEG)
    pd, ns = _sc_scores(users, pos_flat, neg_pairs,
                        _relayout_rows(user_emb), _relayout_rows(item_emb))
    out = _tc_finish(pd.reshape(32, 128), ns.reshape(32, 128))
    return out[0, 0]


# speedup vs baseline: 2.8708x; 1.1129x over previous
import functools

import jax
import jax.numpy as jnp
from jax import lax
from jax.experimental import pallas as pl
from jax.experimental.pallas import tpu as pltpu
from jax.experimental.pallas import tpu_sc as plsc

B = 4096
D = 64
NNEG = 50
TEMP = 0.1
NC = 2
NS = 16
NW = NC * NS
BPW = B // NW
L = 16
NEG_GROUPS = (NNEG + L - 1) // L
PAIRS = BPW // 2


@functools.partial(
    pl.kernel,
    out_type=(
        jax.ShapeDtypeStruct((NW * BPW // L, L), jnp.float32),
        jax.ShapeDtypeStruct((NW * BPW // L, L), jnp.float32),
    ),
    mesh=plsc.VectorSubcoreMesh(core_axis_name="c", subcore_axis_name="s"),
    compiler_params=pltpu.CompilerParams(use_tc_tiling_on_sc=False,
                                         needs_layout_passes=False),
    scratch_types=[
        pltpu.VMEM((BPW,), jnp.int32),
        pltpu.VMEM((BPW,), jnp.int32),
        pltpu.VMEM((PAIRS, 2 * NNEG), jnp.int32),
        pltpu.VMEM((BPW, D), jnp.float32),
        pltpu.VMEM((BPW, D), jnp.float32),
        pltpu.VMEM((2 * NNEG, D), jnp.float32),
        pltpu.VMEM((2 * NNEG, D), jnp.float32),
        pltpu.VMEM((BPW // L, L), jnp.float32),
        pltpu.VMEM((BPW // L, L), jnp.float32),
        pltpu.SemaphoreType.DMA,
        pltpu.SemaphoreType.DMA,
        pltpu.SemaphoreType.DMA,
    ],
)
def _sc_scores(users_hbm, pos_hbm, neg_hbm, uemb_hbm, iemb_hbm,
               pd_out, ns_out,
               uidx, pidx, nidx, urows, prows, nb0, nb1, pd_v, ns_v,
               sem0, sem1, sem2):
    wid = lax.axis_index("s") * NC + lax.axis_index("c")
    base = wid * BPW
    lane = lax.iota(jnp.int32, L)

    pltpu.sync_copy(users_hbm.at[pl.ds(base, BPW)], uidx)
    pltpu.sync_copy(pos_hbm.at[pl.ds(base, BPW)], pidx)
    pltpu.sync_copy(neg_hbm.at[pl.ds(wid * PAIRS, PAIRS)], nidx)

    cu = pltpu.make_async_copy(uemb_hbm.at[uidx], urows, sem2)
    cu.start()
    cp = pltpu.make_async_copy(iemb_hbm.at[pidx], prows, sem2)
    cp.start()
    pltpu.make_async_copy(iemb_hbm.at[nidx.at[0]], nb0, sem0).start()
    pltpu.make_async_copy(iemb_hbm.at[nidx.at[1]], nb1, sem1).start()
    cu.wait()
    cp.wait()

    for r in range(BPW // L):
        pd_v[r] = jnp.zeros((L,), jnp.float32)
        ns_v[r] = jnp.zeros((L,), jnp.float32)

    perms = {w: lane ^ w for w in (8, 4, 2, 1)}
    masks = {w: (lane & w) == 0 for w in (8, 4, 2, 1)}
    gdn = lax.GatherDimensionNumbers(
        offset_dims=(), collapsed_slice_dims=(0,), start_index_map=(0,))

    def _take(v, w):
        return lax.gather(v, perms[w][:, None], dimension_numbers=gdn,
                          slice_sizes=(1,),
                          mode=lax.GatherScatterMode.PROMISE_IN_BOUNDS)

    def _hsum(v):
        for w in (8, 4, 2, 1):
            v = v + _take(v, w)
        return v

    def _butterfly(vecs):
        for w in (8, 4, 2, 1):
            nxt = []
            for i in range(0, len(vecs), 2):
                a, c = vecs[i], vecs[i + 1]
                nxt.append(jnp.where(masks[w], a + _take(a, w), c + _take(c, w)))
            vecs = nxt
        return vecs[0]

    def _dot_parts(u, ref, r):
        dv = u[0] * ref[r, pl.ds(0, L)]
        for j in range(1, D // L):
            dv = dv + u[j] * ref[r, pl.ds(j * L, L)]
        return dv

    def do_b(b, nb, nb_half):
        u = [urows[b, pl.ds(j * L, L)] for j in range(D // L)]

        pdv = _hsum(_dot_parts(u, prows, b))

        pad = jnp.full((L,), -1e29, jnp.float32)
        nacc = jnp.zeros((L,), jnp.float32)
        for g in range(NEG_GROUPS):
            cnt = min(L, NNEG - g * L)
            partials = []
            for k in range(L):
                if k < cnt:
                    partials.append(
                        _dot_parts(u, nb, nb_half * NNEG + g * L + k))
                else:
                    partials.append(pad)
            sv = _butterfly(partials)
            nacc = nacc + jnp.exp(sv / jnp.float32(TEMP))
        nsv = _hsum(nacc)

        row = b // L
        sel = lane == (b % L)
        plsc.addupdate(pd_v.at[row], jnp.where(sel, pdv, jnp.float32(0.0)))
        plsc.addupdate(ns_v.at[row], jnp.where(sel, nsv, jnp.float32(0.0)))

    def do_pair(p, nb, sem):
        pltpu.make_async_copy(iemb_hbm.at[nidx.at[p]], nb, sem).wait()
        do_b(2 * p, nb, 0)
        do_b(2 * p + 1, nb, 1)

        @pl.when(p + 2 < PAIRS)
        def _():
            pltpu.make_async_copy(iemb_hbm.at[nidx.at[p + 2]], nb, sem).start()

    def body(i, carry):
        do_pair(2 * i, nb0, sem0)
        do_pair(2 * i + 1, nb1, sem1)
        return carry

    lax.fori_loop(0, PAIRS // 2, body, 0)

    pltpu.sync_copy(pd_v, pd_out.at[pl.ds(wid * (BPW // L), BPW // L)])
    pltpu.sync_copy(ns_v, ns_out.at[pl.ds(wid * (BPW // L), BPW // L)])


def _tc_finish_body(pd_ref, ns_ref, o_ref):
    s = pd_ref[...] / jnp.float32(TEMP)
    p = jnp.exp(s)
    loss = jnp.log(p + ns_ref[...]) - s
    o_ref[0, 0] = jnp.sum(loss) * jnp.float32(1.0 / B)


_tc_finish = pl.pallas_call(
    _tc_finish_body,
    out_shape=jax.ShapeDtypeStruct((1, 1), jnp.float32),
    out_specs=pl.BlockSpec(memory_space=pltpu.SMEM),
)


TBLK = 16384


def _tp_body(x_ref, o_ref, t_ref):
    t_ref[...] = jnp.swapaxes(x_ref[...], 0, 1)
    e = t_ref[pl.Slice(0, TBLK // 2, 2), :]
    o = t_ref[pl.Slice(1, TBLK // 2, 2), :]
    o_ref[...] = jnp.concatenate([e, o], axis=1)


def _relayout_rows(table):
    n = table.shape[0]
    packed = pl.pallas_call(
        _tp_body,
        grid=((n + TBLK - 1) // TBLK,),
        in_specs=[pl.BlockSpec((D, TBLK), lambda j: (0, j))],
        out_specs=pl.BlockSpec((TBLK // 2, 2 * D), lambda j: (j, 0)),
        out_shape=jax.ShapeDtypeStruct((n // 2, 2 * D), jnp.float32),
        scratch_shapes=[pltpu.VMEM((TBLK, D), jnp.float32)],
    )(table.T)
    return packed.reshape(n, D)


def kernel(users, positives, negatives, epoch, user_emb, item_emb):
    del epoch
    users = users.astype(jnp.int32)
    pos_flat = positives.reshape(B).astype(jnp.int32)
    neg_pairs = negatives.astype(jnp.int32).reshape(B // 2, 2 * NNEG)
    pd, ns = _sc_scores(users, pos_flat, neg_pairs,
                        _relayout_rows(user_emb), _relayout_rows(item_emb))
    out = _tc_finish(pd.reshape(32, 128), ns.reshape(32, 128))
    return out[0, 0]
